# baseline (speedup 1.0000x reference)
import functools

import jax
import jax.numpy as jnp
from jax import lax
from jax.experimental import pallas as pl
from jax.experimental.pallas import tpu as pltpu
from jax.experimental.pallas import tpu_sc as plsc

N_CLASS = 1000000
DROPOUT_PROB = 0.1
SLOT = 128


@functools.lru_cache
def _build(batch: int):
    info = plsc.get_sparse_core_info()
    nc, ns, lanes = info.num_cores, info.num_subcores, info.num_lanes
    nw = nc * ns
    chunk = 128
    assert batch % (nw * chunk) == 0
    b_per_w = batch // nw
    n_chunks = b_per_w // chunk
    mesh = plsc.VectorSubcoreMesh(core_axis_name="c", subcore_axis_name="s")

    @functools.partial(
        pl.kernel,
        mesh=mesh,
        out_type=jax.ShapeDtypeStruct((batch, SLOT), jnp.float32),
        scratch_types=[
            pltpu.VMEM((chunk,), jnp.int32),
            pltpu.VMEM((chunk,), jnp.int32),
            pltpu.VMEM((chunk, SLOT), jnp.float32),
            pltpu.SemaphoreType.DMA,
        ],
    )
    def emb(table_hbm, labels_hbm, mask_hbm, out_hbm, idx_v, msk_v, rows_v, sem):
        wid = lax.axis_index("s") * nc + lax.axis_index("c")
        for q in range(n_chunks):
            base = wid * b_per_w + q * chunk
            pltpu.sync_copy(labels_hbm.at[pl.ds(base, chunk)], idx_v)
            pltpu.sync_copy(mask_hbm.at[pl.ds(base, chunk)], msk_v)
            for j in range(chunk // lanes):
                s = pl.ds(j * lanes, lanes)
                idx_v[s] = jnp.where(msk_v[s] != 0, N_CLASS, idx_v[s])
            pltpu.async_copy(table_hbm.at[idx_v], rows_v, sem).wait()
            pltpu.sync_copy(rows_v, out_hbm.at[pl.ds(base, chunk)])

    return emb


def kernel(labels, table):
    batch = labels.shape[0]
    hidden = table.shape[1]
    u = jax.random.uniform(jax.random.key(1234), (batch,))
    mask = (u < DROPOUT_PROB).astype(jnp.int32)
    padded = jnp.pad(table, ((0, 0), (0, SLOT - hidden)))
    out = _build(batch)(padded, labels.astype(jnp.int32), mask)
    return out[:, :hidden]


# speedup vs baseline: 1.0057x; 1.0057x over previous
import functools

import jax
import jax.numpy as jnp
from jax import lax
from jax.experimental import pallas as pl
from jax.experimental.pallas import tpu as pltpu
from jax.experimental.pallas import tpu_sc as plsc

N_CLASS = 1000000
DROPOUT_PROB = 0.1
SLOT = 128


@functools.lru_cache
def _build(batch: int):
    info = plsc.get_sparse_core_info()
    nc, ns, lanes = info.num_cores, info.num_subcores, info.num_lanes
    nw = nc * ns
    chunk = 128
    assert batch % (nw * chunk) == 0
    b_per_w = batch // nw
    n_chunks = b_per_w // chunk
    mesh = plsc.VectorSubcoreMesh(core_axis_name="c", subcore_axis_name="s")

    @functools.partial(
        pl.kernel,
        mesh=mesh,
        out_type=jax.ShapeDtypeStruct((batch, SLOT), jnp.float32),
        scratch_types=[
            pltpu.VMEM((b_per_w,), jnp.int32),
            pltpu.VMEM((b_per_w,), jnp.int32),
            pltpu.VMEM((b_per_w, SLOT), jnp.float32),
            pltpu.SemaphoreType.DMA,
        ],
    )
    def emb(table_hbm, labels_hbm, mask_hbm, out_hbm, idx_v, msk_v, rows_v, sem):
        wid = lax.axis_index("s") * nc + lax.axis_index("c")
        base = wid * b_per_w
        pltpu.sync_copy(labels_hbm.at[pl.ds(base, b_per_w)], idx_v)
        pltpu.sync_copy(mask_hbm.at[pl.ds(base, b_per_w)], msk_v)
        for j in range(b_per_w // lanes):
            s = pl.ds(j * lanes, lanes)
            idx_v[s] = jnp.where(msk_v[s] != 0, N_CLASS, idx_v[s])
        cps = [
            pltpu.async_copy(
                table_hbm.at[idx_v.at[pl.ds(q * chunk, chunk)]],
                rows_v.at[pl.ds(q * chunk, chunk)],
                sem,
            )
            for q in range(n_chunks)
        ]
        for cp in cps:
            cp.wait()
        pltpu.sync_copy(rows_v, out_hbm.at[pl.ds(base, b_per_w)])

    return emb


def kernel(labels, table):
    batch = labels.shape[0]
    hidden = table.shape[1]
    u = jax.random.uniform(jax.random.key(1234), (batch,))
    mask = (u < DROPOUT_PROB).astype(jnp.int32)
    padded = jnp.pad(table, ((0, 0), (0, SLOT - hidden)))
    out = _build(batch)(padded, labels.astype(jnp.int32), mask)
    return out[:, :hidden]
